# baseline (device time: 14505 ns/iter reference)
import jax
import jax.numpy as jnp
from jax import lax
from jax.experimental import pallas as pl
from jax.experimental.pallas import tpu as pltpu

N_DEV = 4
B, SQ_PER, SKV_PER, HQ, DH = 2, 128, 128, 4, 64
D_MODEL = 512
D_QK = HQ * DH
BLK = 64


def kernel(x, Wq, K_ext, V_ext, Wo):
    def body(x_ref, wq_ref, kt_ref, vt_ref, wo_ref, out_ref,
             kv_ref, send_sems, recv_sems):
        my = lax.axis_index("i")
        partner = (my + 2) % N_DEV

        barrier_sem = pltpu.get_barrier_semaphore()
        pl.semaphore_signal(
            barrier_sem, inc=1,
            device_id=(partner,), device_id_type=pl.DeviceIdType.MESH,
        )

        def send(b):
            r = pltpu.make_async_remote_copy(
                src_ref=kv_ref.at[0, b], dst_ref=kv_ref.at[1, b],
                send_sem=send_sems.at[b], recv_sem=recv_sems.at[b],
                device_id=(partner,), device_id_type=pl.DeviceIdType.MESH,
            )
            r.start()
            return r

        rdmas = []
        for b in range(B):
            kv_ref[0, b, :D_QK, :] = (
                kt_ref[b].astype(jnp.bfloat16).reshape(D_QK, SKV_PER))
            kv_ref[0, b, D_QK:, :] = (
                vt_ref[b].astype(jnp.bfloat16).reshape(D_QK, SKV_PER))
            if b == 0:
                pl.semaphore_wait(barrier_sem, 1)
            rdmas.append(send(b))

        wq = wq_ref[...].astype(jnp.bfloat16)
        xx = x_ref[...].astype(jnp.bfloat16).reshape(B * SQ_PER, D_MODEL)
        q2 = (jnp.dot(xx, wq, preferred_element_type=jnp.float32)
              * 0.125).astype(jnp.bfloat16)

        ctx_rows = []
        for b in range(B):
            rdmas[b].wait_recv()
            for t in range(2):
                r0 = b * SQ_PER + t * BLK
                heads = []
                for h in range(HQ):
                    q = q2[r0:r0 + BLK, h * DH:(h + 1) * DH]
                    krows = pl.ds(h * DH, DH)
                    vrows = pl.ds(D_QK + h * DH, DH)
                    scols = pl.ds(t * BLK, BLK)
                    s_l = jnp.dot(q, kv_ref[0, b, krows, scols],
                                  preferred_element_type=jnp.float32)
                    s_r = jnp.dot(q, kv_ref[1, b, krows, scols],
                                  preferred_element_type=jnp.float32)
                    w_l = jnp.exp(s_l)
                    w_r = jnp.exp(s_r)
                    wsum = (jnp.sum(w_l, axis=-1, keepdims=True)
                            + jnp.sum(w_r, axis=-1, keepdims=True))
                    dn = (((1,), (1,)), ((), ()))
                    ctx = (
                        lax.dot_general(
                            w_l.astype(jnp.bfloat16), kv_ref[0, b, vrows, scols],
                            dimension_numbers=dn,
                            preferred_element_type=jnp.float32)
                        + lax.dot_general(
                            w_r.astype(jnp.bfloat16), kv_ref[1, b, vrows, scols],
                            dimension_numbers=dn,
                            preferred_element_type=jnp.float32)
                    )
                    heads.append((ctx * (1.0 / wsum)).astype(jnp.bfloat16))
                ctx_rows.append(jnp.concatenate(heads, axis=1))
        ctx_all = jnp.concatenate(ctx_rows, axis=0)

        wo = wo_ref[...].astype(jnp.bfloat16)
        out = jnp.dot(ctx_all, wo, preferred_element_type=jnp.float32)
        out_ref[...] = out.reshape(B, SQ_PER, D_MODEL)

        for b in range(B):
            rdmas[b].wait_send()

    K_t = jnp.transpose(K_ext, (0, 2, 3, 1))
    V_t = jnp.transpose(V_ext, (0, 2, 3, 1))

    return pl.pallas_call(
        body,
        out_shape=jax.ShapeDtypeStruct((B, SQ_PER, D_MODEL), jnp.float32),
        in_specs=[pl.BlockSpec(memory_space=pltpu.VMEM)] * 5,
        out_specs=pl.BlockSpec(memory_space=pltpu.VMEM),
        scratch_shapes=[
            pltpu.VMEM((2, B, 2 * D_QK, SKV_PER), jnp.bfloat16),
            pltpu.SemaphoreType.DMA((B,)),
            pltpu.SemaphoreType.DMA((B,)),
        ],
        compiler_params=pltpu.CompilerParams(collective_id=0),
    )(x, Wq, K_t, V_t, Wo)
